# speedup vs baseline: 2.8651x; 2.8651x over previous
import functools

import jax
import jax.numpy as jnp
from jax.experimental import pallas as pl
from jax.experimental.pallas import tpu as pltpu

IN_PLANES = 256
NUM_CLASSES = 128
NUM_MICRO = 64
TEMP = 0.1
ROWS = 512


def _tree_reduce(op, xs):
    xs = list(xs)
    while len(xs) > 1:
        nxt = [op(xs[i], xs[i + 1]) for i in range(0, len(xs) - 1, 2)]
        if len(xs) % 2:
            nxt.append(xs[-1])
        xs = nxt
    return xs[0]


def _cos_proto_kernel(x_ref, p_ref, m_ref, res_ref, idx_ref, pn_ref):
    @pl.when(pl.program_id(0) == 0)
    def _():
        p = p_ref[...]
        n = jnp.sqrt(jnp.sum(p * p, axis=0, keepdims=True))
        pn_ref[...] = p / jnp.maximum(n, 1e-12)

    x = x_ref[...]
    n = jnp.sqrt(jnp.sum(x * x, axis=1, keepdims=True))
    xn = x / jnp.maximum(n, 1e-12)
    sims = jax.lax.dot_general(
        xn, pn_ref[...], (((1,), (0,)), ((), ())),
        preferred_element_type=jnp.float32,
    )

    chunks = [sims[:, k * NUM_CLASSES:(k + 1) * NUM_CLASSES]
              for k in range(NUM_MICRO)]
    per_class = _tree_reduce(jnp.maximum, chunks)
    vmax = jnp.max(per_class, axis=1, keepdims=True)

    lane = jax.lax.broadcasted_iota(jnp.int32, (ROWS, NUM_CLASSES), 1)
    orig_base = lane * NUM_MICRO
    big = jnp.int32(2 ** 30)
    cands = [jnp.where(chunks[k] == vmax, orig_base + k, big)
             for k in range(NUM_MICRO)]
    idx = _tree_reduce(jnp.minimum, cands)
    idx_ref[...] = jnp.min(idx, axis=1, keepdims=True)

    keep = m_ref[...] != 0
    res_ref[...] = jnp.where(keep, per_class * TEMP, 0.0)


def kernel(x, select_mask, proto_list):
    bs, c, h, w = x.shape
    n_rows = bs * c * h * w // IN_PLANES
    x2 = x.reshape(n_rows, IN_PLANES)
    mask2 = select_mask.reshape(n_rows, 1)
    p_t = (proto_list.reshape(NUM_CLASSES, NUM_MICRO, IN_PLANES)
           .transpose(2, 1, 0)
           .reshape(IN_PLANES, NUM_CLASSES * NUM_MICRO))

    grid = (n_rows // ROWS,)
    res2, idx2 = pl.pallas_call(
        _cos_proto_kernel,
        grid=grid,
        in_specs=[
            pl.BlockSpec((ROWS, IN_PLANES), lambda i: (i, 0)),
            pl.BlockSpec((IN_PLANES, NUM_CLASSES * NUM_MICRO), lambda i: (0, 0)),
            pl.BlockSpec((ROWS, 1), lambda i: (i, 0)),
        ],
        out_specs=[
            pl.BlockSpec((ROWS, NUM_CLASSES), lambda i: (i, 0)),
            pl.BlockSpec((ROWS, 1), lambda i: (i, 0)),
        ],
        out_shape=[
            jax.ShapeDtypeStruct((n_rows, NUM_CLASSES), jnp.float32),
            jax.ShapeDtypeStruct((n_rows, 1), jnp.int32),
        ],
        scratch_shapes=[pltpu.VMEM((IN_PLANES, NUM_CLASSES * NUM_MICRO),
                                   jnp.float32)],
    )(x2, p_t, mask2)

    res = res2.reshape(bs, h, w, NUM_CLASSES)
    return (res, idx2.reshape(n_rows))


# speedup vs baseline: 2.8971x; 1.0112x over previous
import functools

import jax
import jax.numpy as jnp
from jax.experimental import pallas as pl
from jax.experimental.pallas import tpu as pltpu

IN_PLANES = 256
NUM_CLASSES = 128
NUM_MICRO = 64
TEMP = 0.1
ROWS = 512


def _tree_reduce(op, xs):
    xs = list(xs)
    while len(xs) > 1:
        nxt = [op(xs[i], xs[i + 1]) for i in range(0, len(xs) - 1, 2)]
        if len(xs) % 2:
            nxt.append(xs[-1])
        xs = nxt
    return xs[0]


def _cos_proto_kernel(x_ref, p_ref, m_ref, res_ref, idx_ref, pn_ref):
    @pl.when(pl.program_id(0) == 0)
    def _():
        p = p_ref[...]
        n = jnp.sqrt(jnp.sum(p * p, axis=0, keepdims=True))
        pn_ref[...] = p / jnp.maximum(n, 1e-12)

    x = x_ref[...]
    n = jnp.sqrt(jnp.sum(x * x, axis=1, keepdims=True))
    xn = x / jnp.maximum(n, 1e-12)

    orig_base = jax.lax.broadcasted_iota(
        jnp.int32, (ROWS, NUM_CLASSES), 1) * NUM_MICRO

    v = jnp.full((ROWS, NUM_CLASSES), -jnp.inf, jnp.float32)
    i = jnp.zeros((ROWS, NUM_CLASSES), jnp.int32)
    for k in range(NUM_MICRO):
        s_k = jax.lax.dot_general(
            xn, pn_ref[:, k * NUM_CLASSES:(k + 1) * NUM_CLASSES],
            (((1,), (0,)), ((), ())),
            preferred_element_type=jnp.float32,
        )
        better = s_k > v
        v = jnp.maximum(v, s_k)
        i = jnp.where(better, orig_base + k, i)

    vmax = jnp.max(v, axis=1, keepdims=True)
    big = jnp.int32(2 ** 30)
    idx = jnp.where(v == vmax, i, big)
    idx_ref[...] = jnp.min(idx, axis=1, keepdims=True)

    keep = m_ref[...] != 0
    res_ref[...] = jnp.where(keep, v * TEMP, 0.0)


def kernel(x, select_mask, proto_list):
    bs, c, h, w = x.shape
    n_rows = bs * c * h * w // IN_PLANES
    x2 = x.reshape(n_rows, IN_PLANES)
    mask2 = select_mask.reshape(n_rows, 1)
    p_t = (proto_list.reshape(NUM_CLASSES, NUM_MICRO, IN_PLANES)
           .transpose(2, 1, 0)
           .reshape(IN_PLANES, NUM_CLASSES * NUM_MICRO))

    grid = (n_rows // ROWS,)
    res2, idx2 = pl.pallas_call(
        _cos_proto_kernel,
        grid=grid,
        in_specs=[
            pl.BlockSpec((ROWS, IN_PLANES), lambda i: (i, 0)),
            pl.BlockSpec((IN_PLANES, NUM_CLASSES * NUM_MICRO), lambda i: (0, 0)),
            pl.BlockSpec((ROWS, 1), lambda i: (i, 0)),
        ],
        out_specs=[
            pl.BlockSpec((ROWS, NUM_CLASSES), lambda i: (i, 0)),
            pl.BlockSpec((ROWS, 1), lambda i: (i, 0)),
        ],
        out_shape=[
            jax.ShapeDtypeStruct((n_rows, NUM_CLASSES), jnp.float32),
            jax.ShapeDtypeStruct((n_rows, 1), jnp.int32),
        ],
        scratch_shapes=[pltpu.VMEM((IN_PLANES, NUM_CLASSES * NUM_MICRO),
                                   jnp.float32)],
    )(x2, p_t, mask2)

    res = res2.reshape(bs, h, w, NUM_CLASSES)
    return (res, idx2.reshape(n_rows))


# speedup vs baseline: 3.0222x; 1.0432x over previous
import jax
import jax.numpy as jnp
from jax.experimental import pallas as pl
from jax.experimental.pallas import tpu as pltpu

IN_PLANES = 256
NUM_CLASSES = 128
NUM_MICRO = 64
TEMP = 0.1
ROWS = 512


def _cos_proto_kernel(x_ref, p_ref, m_ref, res_ref, idx_ref, pn_ref):
    @pl.when(pl.program_id(0) == 0)
    def _():
        p = p_ref[...]
        n = jnp.sqrt(jnp.sum(p * p, axis=1, keepdims=True))
        pn = p / jnp.maximum(n, 1e-12)
        p3 = pn.reshape(NUM_CLASSES, NUM_MICRO, IN_PLANES)
        for k in range(NUM_MICRO):
            pn_ref[k * NUM_CLASSES:(k + 1) * NUM_CLASSES, :] = p3[:, k, :]

    x = x_ref[...]
    n = jnp.sqrt(jnp.sum(x * x, axis=1, keepdims=True))
    xn = x / jnp.maximum(n, 1e-12)

    orig_base = jax.lax.broadcasted_iota(
        jnp.int32, (ROWS, NUM_CLASSES), 1) * NUM_MICRO

    v = jnp.full((ROWS, NUM_CLASSES), -jnp.inf, jnp.float32)
    i = jnp.zeros((ROWS, NUM_CLASSES), jnp.int32)
    for k in range(NUM_MICRO):
        s_k = jax.lax.dot_general(
            xn, pn_ref[k * NUM_CLASSES:(k + 1) * NUM_CLASSES, :],
            (((1,), (1,)), ((), ())),
            preferred_element_type=jnp.float32,
        )
        better = s_k > v
        v = jnp.maximum(v, s_k)
        i = jnp.where(better, orig_base + k, i)

    vmax = jnp.max(v, axis=1, keepdims=True)
    big = jnp.int32(2 ** 30)
    idx = jnp.where(v == vmax, i, big)
    idx_ref[...] = jnp.min(idx, axis=1, keepdims=True)

    keep = m_ref[...] != 0
    res_ref[...] = jnp.where(keep, v * TEMP, 0.0)


def kernel(x, select_mask, proto_list):
    bs, c, h, w = x.shape
    n_rows = bs * c * h * w // IN_PLANES
    x2 = x.reshape(n_rows, IN_PLANES)
    mask2 = select_mask.reshape(n_rows, 1)
    n_protos = proto_list.shape[0]

    grid = (n_rows // ROWS,)
    res2, idx2 = pl.pallas_call(
        _cos_proto_kernel,
        grid=grid,
        in_specs=[
            pl.BlockSpec((ROWS, IN_PLANES), lambda i: (i, 0)),
            pl.BlockSpec((n_protos, IN_PLANES), lambda i: (0, 0)),
            pl.BlockSpec((ROWS, 1), lambda i: (i, 0)),
        ],
        out_specs=[
            pl.BlockSpec((ROWS, NUM_CLASSES), lambda i: (i, 0)),
            pl.BlockSpec((ROWS, 1), lambda i: (i, 0)),
        ],
        out_shape=[
            jax.ShapeDtypeStruct((n_rows, NUM_CLASSES), jnp.float32),
            jax.ShapeDtypeStruct((n_rows, 1), jnp.int32),
        ],
        scratch_shapes=[pltpu.VMEM((n_protos, IN_PLANES), jnp.float32)],
    )(x2, proto_list, mask2)

    res = res2.reshape(bs, h, w, NUM_CLASSES)
    return (res, idx2.reshape(n_rows))


# speedup vs baseline: 4.5636x; 1.5100x over previous
import jax
import jax.numpy as jnp
from jax.experimental import pallas as pl
from jax.experimental.pallas import tpu as pltpu

IN_PLANES = 256
NUM_CLASSES = 128
NUM_MICRO = 64
TEMP = 0.1
ROWS = 512
CPD = 4


def _proto_prep_kernel(p_ref, pn_ref):
    p = p_ref[...]
    n = jnp.sqrt(jnp.sum(p * p, axis=1, keepdims=True))
    pn = p / jnp.maximum(n, 1e-12)
    p3 = pn.reshape(NUM_CLASSES, NUM_MICRO, IN_PLANES)
    for k in range(NUM_MICRO):
        pn_ref[k * NUM_CLASSES:(k + 1) * NUM_CLASSES, :] = p3[:, k, :]


def _cos_proto_kernel(x_ref, pn_ref, m_ref, res_ref, idx_ref):
    x = x_ref[...]
    n = jnp.sqrt(jnp.sum(x * x, axis=1, keepdims=True))
    xn = x / jnp.maximum(n, 1e-12)

    orig_base = jax.lax.broadcasted_iota(
        jnp.int32, (ROWS, NUM_CLASSES), 1) * NUM_MICRO

    v = jnp.full((ROWS, NUM_CLASSES), -jnp.inf, jnp.float32)
    i = jnp.zeros((ROWS, NUM_CLASSES), jnp.int32)
    for t in range(NUM_MICRO // CPD):
        s_t = jax.lax.dot_general(
            xn, pn_ref[t * CPD * NUM_CLASSES:(t + 1) * CPD * NUM_CLASSES, :],
            (((1,), (1,)), ((), ())),
            preferred_element_type=jnp.float32,
        )
        for j in range(CPD):
            k = t * CPD + j
            s_k = s_t[:, j * NUM_CLASSES:(j + 1) * NUM_CLASSES]
            better = s_k > v
            v = jnp.maximum(v, s_k)
            i = jnp.where(better, orig_base + k, i)

    v_t = v.T
    i_t = i.T
    vmax_t = jnp.max(v_t, axis=0, keepdims=True)
    big = jnp.int32(2 ** 30)
    idx_t = jnp.where(v_t == vmax_t, i_t, big)
    idx_ref[...] = jnp.min(idx_t, axis=0, keepdims=True).reshape(1, 1, ROWS)

    keep = m_ref[0].T != 0
    res_ref[...] = jnp.where(keep, v * TEMP, 0.0)


def kernel(x, select_mask, proto_list):
    bs, c, h, w = x.shape
    n_rows = bs * c * h * w // IN_PLANES
    x2 = x.reshape(n_rows, IN_PLANES)
    mask2 = select_mask.reshape(n_rows // ROWS, 1, ROWS)
    n_protos = proto_list.shape[0]

    pn = pl.pallas_call(
        _proto_prep_kernel,
        out_shape=jax.ShapeDtypeStruct((n_protos, IN_PLANES), jnp.float32),
    )(proto_list)

    grid = (n_rows // ROWS,)
    res2, idx2 = pl.pallas_call(
        _cos_proto_kernel,
        grid=grid,
        in_specs=[
            pl.BlockSpec((ROWS, IN_PLANES), lambda i: (i, 0)),
            pl.BlockSpec((n_protos, IN_PLANES), lambda i: (0, 0)),
            pl.BlockSpec((1, 1, ROWS), lambda i: (i, 0, 0)),
        ],
        out_specs=[
            pl.BlockSpec((ROWS, NUM_CLASSES), lambda i: (i, 0)),
            pl.BlockSpec((1, 1, ROWS), lambda i: (i, 0, 0)),
        ],
        out_shape=[
            jax.ShapeDtypeStruct((n_rows, NUM_CLASSES), jnp.float32),
            jax.ShapeDtypeStruct((n_rows // ROWS, 1, ROWS), jnp.int32),
        ],
        compiler_params=pltpu.CompilerParams(
            dimension_semantics=("parallel",)),
    )(x2, pn, mask2)

    res = res2.reshape(bs, h, w, NUM_CLASSES)
    return (res, idx2.reshape(n_rows))


# speedup vs baseline: 4.8235x; 1.0570x over previous
import jax
import jax.numpy as jnp
from jax.experimental import pallas as pl
from jax.experimental.pallas import tpu as pltpu

IN_PLANES = 256
NUM_CLASSES = 128
NUM_MICRO = 64
TEMP = 0.1
ROWS = 1024
CPD = 8


def _proto_prep_kernel(p_ref, pn_ref):
    p = p_ref[...]
    n = jnp.sqrt(jnp.sum(p * p, axis=1, keepdims=True))
    pn = p / jnp.maximum(n, 1e-12)
    p3 = pn.reshape(NUM_CLASSES, NUM_MICRO, IN_PLANES)
    for k in range(NUM_MICRO):
        pn_ref[k * NUM_CLASSES:(k + 1) * NUM_CLASSES, :] = p3[:, k, :]


def _cos_proto_kernel(x_ref, pn_ref, m_ref, res_ref, idx_ref):
    x = x_ref[...]
    n = jnp.sqrt(jnp.sum(x * x, axis=1, keepdims=True))
    xn = x / jnp.maximum(n, 1e-12)

    orig_base = jax.lax.broadcasted_iota(
        jnp.int32, (ROWS, NUM_CLASSES), 1) * NUM_MICRO

    v = jnp.full((ROWS, NUM_CLASSES), -jnp.inf, jnp.float32)
    i = jnp.zeros((ROWS, NUM_CLASSES), jnp.int32)
    for t in range(NUM_MICRO // CPD):
        s_t = jax.lax.dot_general(
            xn, pn_ref[t * CPD * NUM_CLASSES:(t + 1) * CPD * NUM_CLASSES, :],
            (((1,), (1,)), ((), ())),
            preferred_element_type=jnp.float32,
        )
        for j in range(CPD):
            k = t * CPD + j
            s_k = s_t[:, j * NUM_CLASSES:(j + 1) * NUM_CLASSES]
            better = s_k > v
            v = jnp.maximum(v, s_k)
            i = jnp.where(better, orig_base + k, i)

    v_t = v.T
    i_t = i.T
    vmax_t = jnp.max(v_t, axis=0, keepdims=True)
    big = jnp.int32(2 ** 30)
    idx_t = jnp.where(v_t == vmax_t, i_t, big)
    idx_ref[...] = jnp.min(idx_t, axis=0, keepdims=True).reshape(1, 1, ROWS)

    keep = m_ref[0].T != 0
    res_ref[...] = jnp.where(keep, v * TEMP, 0.0)


def kernel(x, select_mask, proto_list):
    bs, c, h, w = x.shape
    n_rows = bs * c * h * w // IN_PLANES
    x2 = x.reshape(n_rows, IN_PLANES)
    mask2 = select_mask.reshape(n_rows // ROWS, 1, ROWS)
    n_protos = proto_list.shape[0]

    pn = pl.pallas_call(
        _proto_prep_kernel,
        out_shape=jax.ShapeDtypeStruct((n_protos, IN_PLANES), jnp.float32),
    )(proto_list)

    grid = (n_rows // ROWS,)
    res2, idx2 = pl.pallas_call(
        _cos_proto_kernel,
        grid=grid,
        in_specs=[
            pl.BlockSpec((ROWS, IN_PLANES), lambda i: (i, 0)),
            pl.BlockSpec((n_protos, IN_PLANES), lambda i: (0, 0)),
            pl.BlockSpec((1, 1, ROWS), lambda i: (i, 0, 0)),
        ],
        out_specs=[
            pl.BlockSpec((ROWS, NUM_CLASSES), lambda i: (i, 0)),
            pl.BlockSpec((1, 1, ROWS), lambda i: (i, 0, 0)),
        ],
        out_shape=[
            jax.ShapeDtypeStruct((n_rows, NUM_CLASSES), jnp.float32),
            jax.ShapeDtypeStruct((n_rows // ROWS, 1, ROWS), jnp.int32),
        ],
        compiler_params=pltpu.CompilerParams(
            dimension_semantics=("parallel",)),
    )(x2, pn, mask2)

    res = res2.reshape(bs, h, w, NUM_CLASSES)
    return (res, idx2.reshape(n_rows))


# speedup vs baseline: 4.9022x; 1.0163x over previous
import jax
import jax.numpy as jnp
from jax.experimental import pallas as pl
from jax.experimental.pallas import tpu as pltpu

IN_PLANES = 256
NUM_CLASSES = 128
NUM_MICRO = 64
TEMP = 0.1
ROWS = 2048
CPD = 8


def _proto_prep_kernel(p_ref, pn_ref):
    p = p_ref[...]
    n = jnp.sqrt(jnp.sum(p * p, axis=1, keepdims=True))
    pn = p / jnp.maximum(n, 1e-12)
    p3 = pn.reshape(NUM_CLASSES, NUM_MICRO, IN_PLANES)
    for k in range(NUM_MICRO):
        pn_ref[k * NUM_CLASSES:(k + 1) * NUM_CLASSES, :] = p3[:, k, :]


def _cos_proto_kernel(x_ref, pn_ref, m_ref, res_ref, idx_ref):
    x = x_ref[...]
    n = jnp.sqrt(jnp.sum(x * x, axis=1, keepdims=True))
    xn = x / jnp.maximum(n, 1e-12)

    orig_base = jax.lax.broadcasted_iota(
        jnp.int32, (ROWS, NUM_CLASSES), 1) * NUM_MICRO

    v = jnp.full((ROWS, NUM_CLASSES), -jnp.inf, jnp.float32)
    i = jnp.zeros((ROWS, NUM_CLASSES), jnp.int32)
    for t in range(NUM_MICRO // CPD):
        s_t = jax.lax.dot_general(
            xn, pn_ref[t * CPD * NUM_CLASSES:(t + 1) * CPD * NUM_CLASSES, :],
            (((1,), (1,)), ((), ())),
            preferred_element_type=jnp.float32,
        )
        for j in range(CPD):
            k = t * CPD + j
            s_k = s_t[:, j * NUM_CLASSES:(j + 1) * NUM_CLASSES]
            better = s_k > v
            v = jnp.maximum(v, s_k)
            i = jnp.where(better, orig_base + k, i)

    v_t = v.T
    i_t = i.T
    vmax_t = jnp.max(v_t, axis=0, keepdims=True)
    big = jnp.int32(2 ** 30)
    idx_t = jnp.where(v_t == vmax_t, i_t, big)
    idx_ref[...] = jnp.min(idx_t, axis=0, keepdims=True).reshape(1, 1, ROWS)

    keep = m_ref[0].T != 0
    res_ref[...] = jnp.where(keep, v * TEMP, 0.0)


def kernel(x, select_mask, proto_list):
    bs, c, h, w = x.shape
    n_rows = bs * c * h * w // IN_PLANES
    x2 = x.reshape(n_rows, IN_PLANES)
    mask2 = select_mask.reshape(n_rows // ROWS, 1, ROWS)
    n_protos = proto_list.shape[0]

    pn = pl.pallas_call(
        _proto_prep_kernel,
        out_shape=jax.ShapeDtypeStruct((n_protos, IN_PLANES), jnp.float32),
    )(proto_list)

    grid = (n_rows // ROWS,)
    res2, idx2 = pl.pallas_call(
        _cos_proto_kernel,
        grid=grid,
        in_specs=[
            pl.BlockSpec((ROWS, IN_PLANES), lambda i: (i, 0)),
            pl.BlockSpec((n_protos, IN_PLANES), lambda i: (0, 0)),
            pl.BlockSpec((1, 1, ROWS), lambda i: (i, 0, 0)),
        ],
        out_specs=[
            pl.BlockSpec((ROWS, NUM_CLASSES), lambda i: (i, 0)),
            pl.BlockSpec((1, 1, ROWS), lambda i: (i, 0, 0)),
        ],
        out_shape=[
            jax.ShapeDtypeStruct((n_rows, NUM_CLASSES), jnp.float32),
            jax.ShapeDtypeStruct((n_rows // ROWS, 1, ROWS), jnp.int32),
        ],
        compiler_params=pltpu.CompilerParams(
            dimension_semantics=("parallel",)),
    )(x2, pn, mask2)

    res = res2.reshape(bs, h, w, NUM_CLASSES)
    return (res, idx2.reshape(n_rows))


# speedup vs baseline: 4.9046x; 1.0005x over previous
import jax
import jax.numpy as jnp
from jax.experimental import pallas as pl
from jax.experimental.pallas import tpu as pltpu

IN_PLANES = 256
NUM_CLASSES = 128
NUM_MICRO = 64
TEMP = 0.1
ROWS = 2048
CPD = 16


def _proto_prep_kernel(p_ref, pn_ref):
    p = p_ref[...]
    n = jnp.sqrt(jnp.sum(p * p, axis=1, keepdims=True))
    pn = p / jnp.maximum(n, 1e-12)
    p3 = pn.reshape(NUM_CLASSES, NUM_MICRO, IN_PLANES)
    for k in range(NUM_MICRO):
        pn_ref[k * NUM_CLASSES:(k + 1) * NUM_CLASSES, :] = p3[:, k, :]


def _cos_proto_kernel(x_ref, pn_ref, m_ref, res_ref, idx_ref):
    x = x_ref[...]
    n = jnp.sqrt(jnp.sum(x * x, axis=1, keepdims=True))
    xn = x / jnp.maximum(n, 1e-12)

    orig_base = jax.lax.broadcasted_iota(
        jnp.int32, (ROWS, NUM_CLASSES), 1) * NUM_MICRO

    v = jnp.full((ROWS, NUM_CLASSES), -jnp.inf, jnp.float32)
    i = jnp.zeros((ROWS, NUM_CLASSES), jnp.int32)
    for t in range(NUM_MICRO // CPD):
        s_t = jax.lax.dot_general(
            xn, pn_ref[t * CPD * NUM_CLASSES:(t + 1) * CPD * NUM_CLASSES, :],
            (((1,), (1,)), ((), ())),
            preferred_element_type=jnp.float32,
        )
        for j in range(CPD):
            k = t * CPD + j
            s_k = s_t[:, j * NUM_CLASSES:(j + 1) * NUM_CLASSES]
            better = s_k > v
            v = jnp.maximum(v, s_k)
            i = jnp.where(better, orig_base + k, i)

    v_t = v.T
    i_t = i.T
    vmax_t = jnp.max(v_t, axis=0, keepdims=True)
    big = jnp.int32(2 ** 30)
    idx_t = jnp.where(v_t == vmax_t, i_t, big)
    idx_ref[...] = jnp.min(idx_t, axis=0, keepdims=True).reshape(1, 1, ROWS)

    keep = m_ref[0].T != 0
    res_ref[...] = jnp.where(keep, v * TEMP, 0.0)


def kernel(x, select_mask, proto_list):
    bs, c, h, w = x.shape
    n_rows = bs * c * h * w // IN_PLANES
    x2 = x.reshape(n_rows, IN_PLANES)
    mask2 = select_mask.reshape(n_rows // ROWS, 1, ROWS)
    n_protos = proto_list.shape[0]

    pn = pl.pallas_call(
        _proto_prep_kernel,
        out_shape=jax.ShapeDtypeStruct((n_protos, IN_PLANES), jnp.float32),
    )(proto_list)

    grid = (n_rows // ROWS,)
    res2, idx2 = pl.pallas_call(
        _cos_proto_kernel,
        grid=grid,
        in_specs=[
            pl.BlockSpec((ROWS, IN_PLANES), lambda i: (i, 0)),
            pl.BlockSpec((n_protos, IN_PLANES), lambda i: (0, 0)),
            pl.BlockSpec((1, 1, ROWS), lambda i: (i, 0, 0)),
        ],
        out_specs=[
            pl.BlockSpec((ROWS, NUM_CLASSES), lambda i: (i, 0)),
            pl.BlockSpec((1, 1, ROWS), lambda i: (i, 0, 0)),
        ],
        out_shape=[
            jax.ShapeDtypeStruct((n_rows, NUM_CLASSES), jnp.float32),
            jax.ShapeDtypeStruct((n_rows // ROWS, 1, ROWS), jnp.int32),
        ],
        compiler_params=pltpu.CompilerParams(
            dimension_semantics=("parallel",)),
    )(x2, pn, mask2)

    res = res2.reshape(bs, h, w, NUM_CLASSES)
    return (res, idx2.reshape(n_rows))
